# baseline (speedup 1.0000x reference)
import dataclasses

import jax
import jax.numpy as jnp
from jax import lax
from jax.experimental import pallas as pl
from jax.experimental.pallas import tpu as pltpu
from jax.experimental.pallas import tpu_sc as plsc


def _sc_compiler_params():
    cp = pltpu.CompilerParams()
    if "needs_layout_passes" in pltpu.CompilerParams.__dataclass_fields__:
        cp = dataclasses.replace(cp, needs_layout_passes=False)
    return cp


N = 50000
D = 128
H = 128
NK = 5
E = 200000
R = 3

BN = 512
NP = 51200
NBLK = NP // BN

NTILE = 16
NW = 32
NT = 64
WIN = 800
EALL = 3 * E
EPAD = 602112
EPT = EPAD // NW
ECH = 2688
K = 128
CAPC = EPT // K + 1
SRC_MASK = (1 << 18) - 1


def _dot(a, b):
    return jnp.dot(a.astype(jnp.bfloat16), b.astype(jnp.bfloat16),
                   preferred_element_type=jnp.float32)


def _pre_body(x_ref, w_ref, o_ref):
    r = pl.program_id(1)
    o_ref[...] = _dot(x_ref[...], w_ref[r])


def _dense_pre(x, w):
    return pl.pallas_call(
        _pre_body,
        grid=(NBLK, R),
        in_specs=[
            pl.BlockSpec((BN, D), lambda i, r: (i, 0)),
            pl.BlockSpec((R, D, H), lambda i, r: (0, 0, 0)),
        ],
        out_specs=pl.BlockSpec((BN, H), lambda i, r: (r * NBLK + i, 0)),
        out_shape=jax.ShapeDtypeStruct((R * NP, H), jnp.float32),
    )(x, w)


def _mid_body(a_ref, lw_ref, lb_ref, w_ref, o_ref, z_ref):
    i = pl.program_id(0)
    r = pl.program_id(1)

    @pl.when(r == 0)
    def _():
        z = _dot(a_ref[...], lw_ref[...]) + lb_ref[...]
        rowid = i * BN + lax.broadcasted_iota(jnp.int32, (BN, H), 0)
        z_ref[...] = jnp.where(rowid < N, jnp.maximum(z, 0.0), 0.0)

    o_ref[...] = _dot(z_ref[...], w_ref[r])


def _dense_mid(agg, lin_w, lin_b, w2):
    return pl.pallas_call(
        _mid_body,
        grid=(NBLK, R),
        in_specs=[
            pl.BlockSpec((BN, H), lambda i, r: (i, 0)),
            pl.BlockSpec((H, H), lambda i, r: (0, 0)),
            pl.BlockSpec((1, H), lambda i, r: (0, 0)),
            pl.BlockSpec((R, H, H), lambda i, r: (0, 0, 0)),
        ],
        out_specs=pl.BlockSpec((BN, H), lambda i, r: (r * NBLK + i, 0)),
        out_shape=jax.ShapeDtypeStruct((R * NP, H), jnp.float32),
        scratch_shapes=[pltpu.VMEM((BN, H), jnp.float32)],
    )(agg, lin_w, lin_b, w2)


def _post_body(a_ref, lw_ref, lb_ref, kw_ref, kb_ref, ab2_ref,
               kn_ref, vt_ref, fw_ref, o_ref):
    z = _dot(a_ref[...], lw_ref[...]) + lb_ref[...]
    z = jnp.maximum(z, 0.0)
    t = _dot(z, kw_ref[...]) + kb_ref[...]
    t = ab2_ref[0:1] * t + ab2_ref[1:2]

    kn = [kn_ref[i:i + 1] for i in range(NK)]
    vt = [vt_ref[i:i + 1] for i in range(NK)]
    h = [kn[i + 1] - kn[i] for i in range(NK - 1)]
    dy = [(vt[i + 1] - vt[i]) / h[i] for i in range(NK - 1)]
    rhs = [6.0 * (dy[i + 1] - dy[i]) for i in range(NK - 2)]
    main = [2.0 * (h[i] + h[i + 1]) for i in range(NK - 2)]
    off = [h[i + 1] for i in range(NK - 3)]
    cp = [off[0] / main[0]]
    dp = [rhs[0] / main[0]]
    for i in range(1, NK - 2):
        denom = main[i] - off[i - 1] * cp[i - 1]
        cp.append(off[i] / denom if i < NK - 3 else None)
        dp.append((rhs[i] - off[i - 1] * dp[i - 1]) / denom)
    m_inner = [None] * (NK - 2)
    m_inner[NK - 3] = dp[NK - 3]
    for i in range(NK - 4, -1, -1):
        m_inner[i] = dp[i] - cp[i] * m_inner[i + 1]
    zero = jnp.zeros_like(vt[0])
    m = [zero] + m_inner + [zero]

    cnt = jnp.zeros_like(t, dtype=jnp.int32)
    for i in range(NK):
        cnt = cnt + (kn[i] < t).astype(jnp.int32)
    idx = jnp.clip(cnt - 1, 0, NK - 2)

    s = jnp.zeros_like(t)
    for i in range(NK - 1):
        msk = (idx == i).astype(jnp.float32)
        hi = h[i]
        aa = kn[i + 1] - t
        bb = t - kn[i]
        seg = ((m[i] * aa ** 3 + m[i + 1] * bb ** 3) / (6.0 * hi)
               + (vt[i] - m[i] * hi ** 2 / 6.0) * aa / hi
               + (vt[i + 1] - m[i + 1] * hi ** 2 / 6.0) * bb / hi)
        s = s + msk * seg

    sb = s.astype(jnp.bfloat16).astype(jnp.float32)
    wb = fw_ref[0:1].astype(jnp.bfloat16).astype(jnp.float32)
    o_ref[...] = (jnp.sum(sb * wb, axis=1, keepdims=True)
                  + fw_ref[1:2, 0:1])


def _dense_post(agg, lin_w, lin_b, kan_w, kan_b, ab2, kn, vt, fw):
    return pl.pallas_call(
        _post_body,
        grid=(NBLK,),
        in_specs=[
            pl.BlockSpec((BN, H), lambda i: (i, 0)),
            pl.BlockSpec((H, H), lambda i: (0, 0)),
            pl.BlockSpec((1, H), lambda i: (0, 0)),
            pl.BlockSpec((H, H), lambda i: (0, 0)),
            pl.BlockSpec((1, H), lambda i: (0, 0)),
            pl.BlockSpec((2, H), lambda i: (0, 0)),
            pl.BlockSpec((NK, H), lambda i: (0, 0)),
            pl.BlockSpec((NK, H), lambda i: (0, 0)),
            pl.BlockSpec((2, H), lambda i: (0, 0)),
        ],
        out_specs=pl.BlockSpec((BN, 1), lambda i: (i, 0)),
        out_shape=jax.ShapeDtypeStruct((NP, 1), jnp.float32),
    )(agg, lin_w, lin_b, kan_w, kan_b, ab2, kn, vt, fw)


def _scan_body(gsrc_hbm, gdst_hbm, spill_hbm, counts_hbm,
               srcb, dstb, spb, cntv, cnts, chks, sem):
    c = lax.axis_index("c")
    s = lax.axis_index("s")
    w = c * NTILE + s
    base_e = w * EPT

    for t in range(NT):
        cnts[t] = 0
        chks[t] = 0

    iota = lax.iota(jnp.int32, 16)

    def flush(t):
        ch = chks[t]
        off = ((w * NT + t) * CAPC + ch) * K
        pltpu.async_copy(
            spb.at[pl.ds(t * 256, K)],
            spill_hbm.at[pl.ds(off, K)], sem).wait()
        chks[t] = ch + 1

    @pl.loop(0, EPT // ECH)
    def _(e):
        off = base_e + e * ECH
        pltpu.async_copy(gsrc_hbm.at[pl.ds(off, ECH)], srcb, sem).wait()
        pltpu.async_copy(gdst_hbm.at[pl.ds(off, ECH)], dstb, sem).wait()

        @pl.loop(0, ECH // 16)
        def _(v):
            dst = dstb[pl.ds(v * 16, 16)]
            src = srcb[pl.ds(v * 16, 16)]
            q = dst >> 5
            u = (q * 5243) >> 17
            loc = dst - u * WIN
            packed = src | (loc << 18)
            for t in range(NT):
                msk = u == t

                @pl.when(jnp.any(msk))
                def _(msk=msk, t=t):
                    npop = lax.reduce_max(
                        plsc.all_reduce_population_count(msk), axes=(0,))
                    mi = msk.astype(jnp.int32)
                    rank = plsc.cumsum(mi) - mi
                    cnt = cnts[t]
                    base = t * 256
                    pos = jnp.where(msk, base + cnt + rank,
                                    base + 240 + (iota & 15))
                    plsc.store_scatter(spb, [pos], packed)
                    cnt_new = cnt + npop

                    @pl.when(cnt_new >= K)
                    def _():
                        flush(t)
                        lv = spb[pl.ds(t * 256 + K, 16)]
                        spb[pl.ds(t * 256, 16)] = lv

                    cnts[t] = jnp.where(cnt_new >= K, cnt_new - K, cnt_new)

    trash = (iota << 18) | (N + iota * 4)
    for t in range(NT):
        cnt = cnts[t]
        for j in range(K // 16):
            spb[pl.ds(t * 256 + cnt + j * 16, 16)] = trash
        flush(t)

    for g in range(NT // 16):
        v = jnp.zeros((16,), jnp.int32)
        for t in range(16):
            v = jnp.where(iota == t, chks[g * 16 + t], v)
        cntv[pl.ds(g * 16, 16)] = v
    pltpu.async_copy(cntv, counts_hbm.at[pl.ds(w * NT, NT)], sem).wait()


def _sc_scan(gsrc, gdst):
    mesh = plsc.VectorSubcoreMesh(
        core_axis_name="c", subcore_axis_name="s",
        num_cores=2, num_subcores=NTILE)
    kern = pl.kernel(
        _scan_body,
        compiler_params=_sc_compiler_params(),
        out_type=(
            jax.ShapeDtypeStruct((NW * NT * CAPC * K + 8 * K,), jnp.int32),
            jax.ShapeDtypeStruct((NW * NT,), jnp.int32),
        ),
        mesh=mesh,
        scratch_types=[
            pltpu.VMEM((ECH,), jnp.int32),
            pltpu.VMEM((ECH,), jnp.int32),
            pltpu.VMEM((NT * 256,), jnp.int32),
            pltpu.VMEM((NT,), jnp.int32),
            pltpu.SMEM((NT,), jnp.int32),
            pltpu.SMEM((NT,), jnp.int32),
            pltpu.SemaphoreType.DMA,
        ],
    )
    return kern(gsrc, gdst)


def _drain_body(h_hbm, spill_hbm, counts_hbm, o_hbm,
                cbuf, slab0, slab1, gb, sb, rb, zb, acc,
                sem_c, sem_l0, sem_l1, sem_g0, sem_g1, sem_g2, sem_g3,
                sem_s0, sem_s1, sem_s2, sem_s3):
    c = lax.axis_index("c")
    s = lax.axis_index("s")
    iota = lax.iota(jnp.int32, 16)
    zv = jnp.zeros((16,), jnp.float32)
    SLABC = 8

    pltpu.async_copy(counts_hbm, cbuf, sem_c).wait()

    @pl.loop(0, 32)
    def _(i):
        for j in range(H // 16):
            zb[i, pl.ds(j * 16, 16)] = zv

    gsems = (sem_g0, sem_g1, sem_g2, sem_g3)
    ssems = (sem_s0, sem_s1, sem_s2, sem_s3)

    @pl.loop(0, NT // 2)
    def _(p):
        u_win = c * (NT // 2) + p
        umask = iota == (u_win & 15)
        usel = (u_win >> 4) * 16

        pltpu.sync_copy(zb, acc.at[pl.ds(s * 32, 32)])

        @pl.when(s < 9)
        def _():
            pltpu.sync_copy(zb, acc.at[pl.ds((16 + s) * 32, 32)])

        plsc.subcore_barrier()

        def nch_of(wp):
            v = cbuf[pl.ds(wp * NT + usel, 16)]
            return lax.reduce_max(jnp.where(umask, v, 0), axes=(0,))

        nch0 = nch_of(s * 2)
        nch1 = nch_of(s * 2 + 1)
        lb0 = ((s * 2) * NT + u_win) * CAPC
        lb1 = ((s * 2 + 1) * NT + u_win) * CAPC

        def unpack(slab, slot, k):
            for j in range(K // 16):
                pv = slab[pl.ds(slot * K + j * 16, 16)]
                gb[k, pl.ds(j * 16, 16)] = pv & SRC_MASK
                sb[k, pl.ds(j * 16, 16)] = pv >> 18

        def do_slab(slab, sn, nch, lsem):
            for half in range(SLABC // 4):
                gds = [None] * 4
                for k in range(4):
                    slot = half * 4 + k
                    cond = sn * SLABC + slot < nch

                    @pl.when(cond)
                    def _(slot=slot, k=k):
                        unpack(slab, slot, k)
                        gds[k] = pltpu.async_copy(
                            h_hbm.at[gb.at[k]], rb.at[k], gsems[k])
                sds = [None] * 4
                for k in range(4):
                    slot = half * 4 + k
                    cond = sn * SLABC + slot < nch

                    @pl.when(cond)
                    def _(slot=slot, k=k):
                        gds[k].wait()
                        sds[k] = pltpu.async_copy(
                            rb.at[k], acc.at[sb.at[k]], ssems[k], add=True)
                for k in range(4):
                    slot = half * 4 + k
                    cond = sn * SLABC + slot < nch

                    @pl.when(cond)
                    def _(slot=slot, k=k):
                        sds[k].wait()

        def do_list(slab, lbase, nch, lsem):
            def body(sn, _):
                pltpu.async_copy(
                    spill_hbm.at[pl.ds((lbase + sn * SLABC) * K,
                                       SLABC * K)],
                    slab, lsem).wait()
                do_slab(slab, sn, nch, lsem)
                return 0

            lax.fori_loop(0, (nch + SLABC - 1) // SLABC, body, 0)

        do_list(slab0, lb0, nch0, sem_l0)
        do_list(slab1, lb1, nch1, sem_l1)

        plsc.subcore_barrier()

        obase = u_win * WIN
        pltpu.sync_copy(acc.at[pl.ds(s * 32, 32)],
                        o_hbm.at[pl.ds(obase + s * 32, 32)])

        @pl.when(s < 9)
        def _():
            pltpu.sync_copy(
                acc.at[pl.ds((16 + s) * 32, 32)],
                o_hbm.at[pl.ds(obase + (16 + s) * 32, 32)])

        plsc.subcore_barrier()


def _sc_drain(hcat, spill, counts):
    mesh = plsc.VectorSubcoreMesh(
        core_axis_name="c", subcore_axis_name="s",
        num_cores=2, num_subcores=NTILE)
    kern = pl.kernel(
        _drain_body,
        compiler_params=_sc_compiler_params(),
        out_type=jax.ShapeDtypeStruct((NP, H), jnp.float32),
        mesh=mesh,
        scratch_types=[
            pltpu.VMEM((NW * NT,), jnp.int32),
            pltpu.VMEM((8 * K,), jnp.int32),
            pltpu.VMEM((8 * K,), jnp.int32),
            pltpu.VMEM((4, K), jnp.int32),
            pltpu.VMEM((4, K), jnp.int32),
            pltpu.VMEM((4, K, H), jnp.float32),
            pltpu.VMEM((32, H), jnp.float32),
            pltpu.VMEM_SHARED((WIN, H), jnp.float32),
        ] + [pltpu.SemaphoreType.DMA] * 11,
    )
    return kern(hcat, spill, counts)


def kernel(x, edge_index_A1, edge_index_A2, edge_index_A3, W1, lin1_w,
           lin1_b, W2, lin2_w, lin2_b, kan_w, kan_b, alpha, beta, knots,
           values, fc_w, fc_b):
    f32 = jnp.float32
    pad_e = EPAD - EALL
    pad_src = (jnp.arange(pad_e, dtype=jnp.int32) * 29) % N
    pad_dst = jnp.full((pad_e,), 1 << 20, dtype=jnp.int32)
    gsrc = jnp.concatenate([
        edge_index_A1[1],
        edge_index_A2[1] + NP,
        edge_index_A3[1] + 2 * NP,
        pad_src,
    ])
    gdst = jnp.concatenate(
        [edge_index_A1[0], edge_index_A2[0], edge_index_A3[0], pad_dst])

    xp = jnp.pad(x.astype(f32), ((0, NP - N), (0, 0)))
    ones = jnp.ones((1, H), f32)
    lb1 = lin1_b.reshape(1, H).astype(f32)
    lb2 = lin2_b.reshape(1, H).astype(f32)
    kb = kan_b.reshape(1, H).astype(f32)
    ab2 = jnp.concatenate(
        [alpha.reshape(1, H), beta.reshape(1, H)]).astype(f32)
    kn = knots.astype(f32).reshape(NK, 1) * ones
    vt = values.astype(f32).T
    fw = jnp.concatenate(
        [fc_w.reshape(1, H), fc_b.reshape(1, 1) * ones]).astype(f32)

    spill, counts = _sc_scan(gsrc, gdst)
    h1 = _dense_pre(xp, W1.astype(f32))
    a1 = _sc_drain(h1, spill, counts)
    h2 = _dense_mid(a1, lin1_w.astype(f32), lb1, W2.astype(f32))
    a2 = _sc_drain(h2, spill, counts)
    out = _dense_post(a2, lin2_w.astype(f32), lb2, kan_w.astype(f32),
                      kb, ab2, kn, vt, fw)
    return out[:N, 0]


# speedup vs baseline: 1.2842x; 1.2842x over previous
import dataclasses

import jax
import jax.numpy as jnp
from jax import lax
from jax.experimental import pallas as pl
from jax.experimental.pallas import tpu as pltpu
from jax.experimental.pallas import tpu_sc as plsc


def _sc_compiler_params():
    cp = pltpu.CompilerParams()
    if "needs_layout_passes" in pltpu.CompilerParams.__dataclass_fields__:
        cp = dataclasses.replace(cp, needs_layout_passes=False)
    return cp


N = 50000
D = 128
H = 128
NK = 5
E = 200000
R = 3

BN = 512
NP = 51200
NBLK = NP // BN

NTILE = 16
NW = 32
NT = 64
WIN = 800
EALL = 3 * E
EPAD = 602112
EPT = EPAD // NW
ECH = 2688
K = 128
CAPC = EPT // K + 1
SRC_MASK = (1 << 18) - 1


def _dot(a, b):
    return jnp.dot(a.astype(jnp.bfloat16), b.astype(jnp.bfloat16),
                   preferred_element_type=jnp.float32)


def _pre_body(x_ref, w_ref, o_ref):
    r = pl.program_id(1)
    o_ref[...] = _dot(x_ref[...], w_ref[r])


def _dense_pre(x, w):
    return pl.pallas_call(
        _pre_body,
        grid=(NBLK, R),
        in_specs=[
            pl.BlockSpec((BN, D), lambda i, r: (i, 0)),
            pl.BlockSpec((R, D, H), lambda i, r: (0, 0, 0)),
        ],
        out_specs=pl.BlockSpec((BN, H), lambda i, r: (r * NBLK + i, 0)),
        out_shape=jax.ShapeDtypeStruct((R * NP, H), jnp.float32),
    )(x, w)


def _mid_body(a_ref, lw_ref, lb_ref, w_ref, o_ref, z_ref):
    i = pl.program_id(0)
    r = pl.program_id(1)

    @pl.when(r == 0)
    def _():
        z = _dot(a_ref[...], lw_ref[...]) + lb_ref[...]
        rowid = i * BN + lax.broadcasted_iota(jnp.int32, (BN, H), 0)
        z_ref[...] = jnp.where(rowid < N, jnp.maximum(z, 0.0), 0.0)

    o_ref[...] = _dot(z_ref[...], w_ref[r])


def _dense_mid(agg, lin_w, lin_b, w2):
    return pl.pallas_call(
        _mid_body,
        grid=(NBLK, R),
        in_specs=[
            pl.BlockSpec((BN, H), lambda i, r: (i, 0)),
            pl.BlockSpec((H, H), lambda i, r: (0, 0)),
            pl.BlockSpec((1, H), lambda i, r: (0, 0)),
            pl.BlockSpec((R, H, H), lambda i, r: (0, 0, 0)),
        ],
        out_specs=pl.BlockSpec((BN, H), lambda i, r: (r * NBLK + i, 0)),
        out_shape=jax.ShapeDtypeStruct((R * NP, H), jnp.float32),
        scratch_shapes=[pltpu.VMEM((BN, H), jnp.float32)],
    )(agg, lin_w, lin_b, w2)


def _post_body(a_ref, lw_ref, lb_ref, kw_ref, kb_ref, ab2_ref,
               kn_ref, vt_ref, fw_ref, o_ref):
    z = _dot(a_ref[...], lw_ref[...]) + lb_ref[...]
    z = jnp.maximum(z, 0.0)
    t = _dot(z, kw_ref[...]) + kb_ref[...]
    t = ab2_ref[0:1] * t + ab2_ref[1:2]

    kn = [kn_ref[i:i + 1] for i in range(NK)]
    vt = [vt_ref[i:i + 1] for i in range(NK)]
    h = [kn[i + 1] - kn[i] for i in range(NK - 1)]
    dy = [(vt[i + 1] - vt[i]) / h[i] for i in range(NK - 1)]
    rhs = [6.0 * (dy[i + 1] - dy[i]) for i in range(NK - 2)]
    main = [2.0 * (h[i] + h[i + 1]) for i in range(NK - 2)]
    off = [h[i + 1] for i in range(NK - 3)]
    cp = [off[0] / main[0]]
    dp = [rhs[0] / main[0]]
    for i in range(1, NK - 2):
        denom = main[i] - off[i - 1] * cp[i - 1]
        cp.append(off[i] / denom if i < NK - 3 else None)
        dp.append((rhs[i] - off[i - 1] * dp[i - 1]) / denom)
    m_inner = [None] * (NK - 2)
    m_inner[NK - 3] = dp[NK - 3]
    for i in range(NK - 4, -1, -1):
        m_inner[i] = dp[i] - cp[i] * m_inner[i + 1]
    zero = jnp.zeros_like(vt[0])
    m = [zero] + m_inner + [zero]

    cnt = jnp.zeros_like(t, dtype=jnp.int32)
    for i in range(NK):
        cnt = cnt + (kn[i] < t).astype(jnp.int32)
    idx = jnp.clip(cnt - 1, 0, NK - 2)

    s = jnp.zeros_like(t)
    for i in range(NK - 1):
        msk = (idx == i).astype(jnp.float32)
        hi = h[i]
        aa = kn[i + 1] - t
        bb = t - kn[i]
        seg = ((m[i] * aa ** 3 + m[i + 1] * bb ** 3) / (6.0 * hi)
               + (vt[i] - m[i] * hi ** 2 / 6.0) * aa / hi
               + (vt[i + 1] - m[i + 1] * hi ** 2 / 6.0) * bb / hi)
        s = s + msk * seg

    sb = s.astype(jnp.bfloat16).astype(jnp.float32)
    wb = fw_ref[0:1].astype(jnp.bfloat16).astype(jnp.float32)
    o_ref[...] = (jnp.sum(sb * wb, axis=1, keepdims=True)
                  + fw_ref[1:2, 0:1])


def _dense_post(agg, lin_w, lin_b, kan_w, kan_b, ab2, kn, vt, fw):
    return pl.pallas_call(
        _post_body,
        grid=(NBLK,),
        in_specs=[
            pl.BlockSpec((BN, H), lambda i: (i, 0)),
            pl.BlockSpec((H, H), lambda i: (0, 0)),
            pl.BlockSpec((1, H), lambda i: (0, 0)),
            pl.BlockSpec((H, H), lambda i: (0, 0)),
            pl.BlockSpec((1, H), lambda i: (0, 0)),
            pl.BlockSpec((2, H), lambda i: (0, 0)),
            pl.BlockSpec((NK, H), lambda i: (0, 0)),
            pl.BlockSpec((NK, H), lambda i: (0, 0)),
            pl.BlockSpec((2, H), lambda i: (0, 0)),
        ],
        out_specs=pl.BlockSpec((BN, 1), lambda i: (i, 0)),
        out_shape=jax.ShapeDtypeStruct((NP, 1), jnp.float32),
    )(agg, lin_w, lin_b, kan_w, kan_b, ab2, kn, vt, fw)


def _scan_body(gsrc_hbm, gdst_hbm, spill_hbm, counts_hbm,
               srcb, dstb, spb, cntv, cnts, chks, sem):
    c = lax.axis_index("c")
    s = lax.axis_index("s")
    w = c * NTILE + s
    base_e = w * EPT

    for t in range(NT):
        cnts[t] = 0
        chks[t] = 0

    iota = lax.iota(jnp.int32, 16)

    def flush(t):
        ch = chks[t]
        off = ((w * NT + t) * CAPC + ch) * K
        pltpu.async_copy(
            spb.at[pl.ds(t * 256, K)],
            spill_hbm.at[pl.ds(off, K)], sem).wait()
        chks[t] = ch + 1

    @pl.loop(0, EPT // ECH)
    def _(e):
        off = base_e + e * ECH
        pltpu.async_copy(gsrc_hbm.at[pl.ds(off, ECH)], srcb, sem).wait()
        pltpu.async_copy(gdst_hbm.at[pl.ds(off, ECH)], dstb, sem).wait()

        @pl.loop(0, ECH // 16)
        def _(v):
            dst = dstb[pl.ds(v * 16, 16)]
            src = srcb[pl.ds(v * 16, 16)]
            q = dst >> 5
            u = (q * 5243) >> 17
            loc = dst - u * WIN
            packed = src | (loc << 18)
            for t in range(NT):
                msk = u == t
                npop = lax.reduce_max(
                    plsc.all_reduce_population_count(msk), axes=(0,))

                @pl.when(npop > 0)
                def _(msk=msk, npop=npop, t=t):
                    mi = msk.astype(jnp.int32)
                    rank = plsc.cumsum(mi) - mi
                    cnt = cnts[t]
                    base = t * 256
                    pos = jnp.where(msk, base + cnt + rank,
                                    base + 240 + (iota & 15))
                    plsc.store_scatter(spb, [pos], packed)
                    cnt_new = cnt + npop

                    @pl.when(cnt_new >= K)
                    def _():
                        flush(t)
                        lv = spb[pl.ds(t * 256 + K, 16)]
                        spb[pl.ds(t * 256, 16)] = lv

                    cnts[t] = jnp.where(cnt_new >= K, cnt_new - K, cnt_new)

    trash = (iota << 18) | (N + iota * 4)
    for t in range(NT):
        cnt = cnts[t]
        for j in range(K // 16):
            spb[pl.ds(t * 256 + cnt + j * 16, 16)] = trash
        flush(t)

    for g in range(NT // 16):
        v = jnp.zeros((16,), jnp.int32)
        for t in range(16):
            v = jnp.where(iota == t, chks[g * 16 + t], v)
        cntv[pl.ds(g * 16, 16)] = v
    pltpu.async_copy(cntv, counts_hbm.at[pl.ds(w * NT, NT)], sem).wait()


def _sc_scan(gsrc, gdst):
    mesh = plsc.VectorSubcoreMesh(
        core_axis_name="c", subcore_axis_name="s",
        num_cores=2, num_subcores=NTILE)
    kern = pl.kernel(
        _scan_body,
        compiler_params=_sc_compiler_params(),
        out_type=(
            jax.ShapeDtypeStruct((NW * NT * CAPC * K + 8 * K,), jnp.int32),
            jax.ShapeDtypeStruct((NW * NT,), jnp.int32),
        ),
        mesh=mesh,
        scratch_types=[
            pltpu.VMEM((ECH,), jnp.int32),
            pltpu.VMEM((ECH,), jnp.int32),
            pltpu.VMEM((NT * 256,), jnp.int32),
            pltpu.VMEM((NT,), jnp.int32),
            pltpu.SMEM((NT,), jnp.int32),
            pltpu.SMEM((NT,), jnp.int32),
            pltpu.SemaphoreType.DMA,
        ],
    )
    return kern(gsrc, gdst)


def _drain_body(h_hbm, spill_hbm, counts_hbm, o_hbm,
                cbuf, slab0, slab1, gb, sb, rb, zb, acc,
                sem_c, sem_l0, sem_l1, sem_g0, sem_g1, sem_g2, sem_g3,
                sem_s0, sem_s1, sem_s2, sem_s3):
    c = lax.axis_index("c")
    s = lax.axis_index("s")
    iota = lax.iota(jnp.int32, 16)
    zv = jnp.zeros((16,), jnp.float32)
    SLABC = 8

    pltpu.async_copy(counts_hbm, cbuf, sem_c).wait()

    @pl.loop(0, 32)
    def _(i):
        for j in range(H // 16):
            zb[i, pl.ds(j * 16, 16)] = zv

    gsems = (sem_g0, sem_g1, sem_g2, sem_g3)
    ssems = (sem_s0, sem_s1, sem_s2, sem_s3)

    @pl.loop(0, NT // 2)
    def _(p):
        u_win = c * (NT // 2) + p
        umask = iota == (u_win & 15)
        usel = (u_win >> 4) * 16

        pltpu.sync_copy(zb, acc.at[pl.ds(s * 32, 32)])

        @pl.when(s < 9)
        def _():
            pltpu.sync_copy(zb, acc.at[pl.ds((16 + s) * 32, 32)])

        plsc.subcore_barrier()

        def nch_of(wp):
            v = cbuf[pl.ds(wp * NT + usel, 16)]
            return lax.reduce_max(jnp.where(umask, v, 0), axes=(0,))

        nch0 = nch_of(s * 2)
        nch1 = nch_of(s * 2 + 1)
        lb0 = ((s * 2) * NT + u_win) * CAPC
        lb1 = ((s * 2 + 1) * NT + u_win) * CAPC

        def unpack(slab, slot, k):
            for j in range(K // 16):
                pv = slab[pl.ds(slot * K + j * 16, 16)]
                gb[k, pl.ds(j * 16, 16)] = pv & SRC_MASK
                sb[k, pl.ds(j * 16, 16)] = pv >> 18

        def do_slab(slab, sn, nch, lsem):
            for half in range(SLABC // 4):
                gds = [None] * 4
                for k in range(4):
                    slot = half * 4 + k
                    cond = sn * SLABC + slot < nch

                    @pl.when(cond)
                    def _(slot=slot, k=k):
                        unpack(slab, slot, k)
                        gds[k] = pltpu.async_copy(
                            h_hbm.at[gb.at[k]], rb.at[k], gsems[k])
                sds = [None] * 4
                for k in range(4):
                    slot = half * 4 + k
                    cond = sn * SLABC + slot < nch

                    @pl.when(cond)
                    def _(slot=slot, k=k):
                        gds[k].wait()
                        sds[k] = pltpu.async_copy(
                            rb.at[k], acc.at[sb.at[k]], ssems[k], add=True)
                for k in range(4):
                    slot = half * 4 + k
                    cond = sn * SLABC + slot < nch

                    @pl.when(cond)
                    def _(slot=slot, k=k):
                        sds[k].wait()

        def do_list(slab, lbase, nch, lsem):
            def body(sn, _):
                pltpu.async_copy(
                    spill_hbm.at[pl.ds((lbase + sn * SLABC) * K,
                                       SLABC * K)],
                    slab, lsem).wait()
                do_slab(slab, sn, nch, lsem)
                return 0

            lax.fori_loop(0, (nch + SLABC - 1) // SLABC, body, 0)

        do_list(slab0, lb0, nch0, sem_l0)
        do_list(slab1, lb1, nch1, sem_l1)

        plsc.subcore_barrier()

        obase = u_win * WIN
        pltpu.sync_copy(acc.at[pl.ds(s * 32, 32)],
                        o_hbm.at[pl.ds(obase + s * 32, 32)])

        @pl.when(s < 9)
        def _():
            pltpu.sync_copy(
                acc.at[pl.ds((16 + s) * 32, 32)],
                o_hbm.at[pl.ds(obase + (16 + s) * 32, 32)])

        plsc.subcore_barrier()


def _sc_drain(hcat, spill, counts):
    mesh = plsc.VectorSubcoreMesh(
        core_axis_name="c", subcore_axis_name="s",
        num_cores=2, num_subcores=NTILE)
    kern = pl.kernel(
        _drain_body,
        compiler_params=_sc_compiler_params(),
        out_type=jax.ShapeDtypeStruct((NP, H), jnp.float32),
        mesh=mesh,
        scratch_types=[
            pltpu.VMEM((NW * NT,), jnp.int32),
            pltpu.VMEM((8 * K,), jnp.int32),
            pltpu.VMEM((8 * K,), jnp.int32),
            pltpu.VMEM((4, K), jnp.int32),
            pltpu.VMEM((4, K), jnp.int32),
            pltpu.VMEM((4, K, H), jnp.float32),
            pltpu.VMEM((32, H), jnp.float32),
            pltpu.VMEM_SHARED((WIN, H), jnp.float32),
        ] + [pltpu.SemaphoreType.DMA] * 11,
    )
    return kern(hcat, spill, counts)


def kernel(x, edge_index_A1, edge_index_A2, edge_index_A3, W1, lin1_w,
           lin1_b, W2, lin2_w, lin2_b, kan_w, kan_b, alpha, beta, knots,
           values, fc_w, fc_b):
    f32 = jnp.float32
    pad_e = EPAD - EALL
    pad_src = (jnp.arange(pad_e, dtype=jnp.int32) * 29) % N
    pad_dst = jnp.full((pad_e,), 1 << 20, dtype=jnp.int32)
    gsrc = jnp.concatenate([
        edge_index_A1[1],
        edge_index_A2[1] + NP,
        edge_index_A3[1] + 2 * NP,
        pad_src,
    ])
    gdst = jnp.concatenate(
        [edge_index_A1[0], edge_index_A2[0], edge_index_A3[0], pad_dst])

    xp = jnp.pad(x.astype(f32), ((0, NP - N), (0, 0)))
    ones = jnp.ones((1, H), f32)
    lb1 = lin1_b.reshape(1, H).astype(f32)
    lb2 = lin2_b.reshape(1, H).astype(f32)
    kb = kan_b.reshape(1, H).astype(f32)
    ab2 = jnp.concatenate(
        [alpha.reshape(1, H), beta.reshape(1, H)]).astype(f32)
    kn = knots.astype(f32).reshape(NK, 1) * ones
    vt = values.astype(f32).T
    fw = jnp.concatenate(
        [fc_w.reshape(1, H), fc_b.reshape(1, 1) * ones]).astype(f32)

    spill, counts = _sc_scan(gsrc, gdst)
    h1 = _dense_pre(xp, W1.astype(f32))
    a1 = _sc_drain(h1, spill, counts)
    h2 = _dense_mid(a1, lin1_w.astype(f32), lb1, W2.astype(f32))
    a2 = _sc_drain(h2, spill, counts)
    out = _dense_post(a2, lin2_w.astype(f32), lb2, kan_w.astype(f32),
                      kb, ab2, kn, vt, fw)
    return out[:N, 0]
